# baseline (speedup 1.0000x reference)
import jax
import jax.numpy as jnp
from jax.experimental import pallas as pl
from jax.experimental.pallas import tpu as pltpu

_NEG = -2.3819763e+38


def _chunks_for(s):
    chunks, rem = [], s
    for c in (16, 16, 32, 64, 128, 256):
        if rem <= 0:
            break
        c = min(c, rem)
        chunks.append(c)
        rem -= c
    while rem > 0:
        c = min(512, rem)
        chunks.append(c)
        rem -= c
    return tuple(chunks)


def _mask_kernel(o_ref, scratch, sems):
    s = scratch.shape[1]
    base = 0
    for k, br in enumerate(_chunks_for(scratch.shape[0])):
        rows = jax.lax.broadcasted_iota(jnp.int32, (br, s), 0) + base
        cols = jax.lax.broadcasted_iota(jnp.int32, (br, s), 1)
        scratch[pl.ds(base, br), :] = jnp.where(cols > rows, _NEG, 0.0).astype(
            jnp.float32
        )
        pltpu.make_async_copy(
            scratch.at[pl.ds(base, br), :],
            o_ref.at[0, 0, pl.ds(base, br), :],
            sems.at[k],
        ).start()
        base += br
    base = 0
    for k, br in enumerate(_chunks_for(scratch.shape[0])):
        pltpu.make_async_copy(
            scratch.at[pl.ds(base, br), :],
            o_ref.at[0, 0, pl.ds(base, br), :],
            sems.at[k],
        ).wait()
        base += br


def kernel(forward, batch_size, data, device, temperature, top_p, top_k, kv_caches, output_len, is_str_prompt):
    S = data.shape[-2]
    return pl.pallas_call(
        _mask_kernel,
        out_specs=pl.BlockSpec(memory_space=pl.ANY),
        out_shape=jax.ShapeDtypeStruct((1, 1, S, S), jnp.float32),
        scratch_shapes=[
            pltpu.VMEM((S, S), jnp.float32),
            pltpu.SemaphoreType.DMA((len(_chunks_for(S)),)),
        ],
    )()
